# baseline (speedup 1.0000x reference)
import math

import jax
import jax.numpy as jnp
import numpy as np
from jax.experimental import pallas as pl
from jax.experimental.pallas import tpu as pltpu

L = 2048
D = 768
H = 12
DK = 64
U = 40
NQB = 8
QB = L // NQB
EPS = 1e-6
NEG = -3.4e38

NP = 3 * H // 2
HP = H // 2
SB0 = NP
SC_STEP = NP + NQB
SD0 = SC_STEP + 1
NSTEPS = SD0 + HP

INTERPRET = False


def _threefry2x32(k0, k1, x0, x1):
    def rotl(x, d):
        return ((x << np.uint32(d)) | (x >> np.uint32(32 - d))).astype(np.uint32)
    rot = [13, 15, 26, 6, 17, 29, 16, 24]
    ks = [np.uint32(k0), np.uint32(k1),
          np.uint32(k0) ^ np.uint32(k1) ^ np.uint32(0x1BD11BDA)]
    x0 = (x0 + ks[0]).astype(np.uint32)
    x1 = (x1 + ks[1]).astype(np.uint32)
    for g in range(5):
        for j in range(4):
            x0 = (x0 + x1).astype(np.uint32)
            x1 = rotl(x1, rot[(g * 4 + j) % 8])
            x1 = x1 ^ x0
        x0 = (x0 + ks[(g + 1) % 3]).astype(np.uint32)
        x1 = (x1 + ks[(g + 2) % 3] + np.uint32(g + 1)).astype(np.uint32)
    return x0, x1


def _sample_counts() -> np.ndarray:
    c0, c1 = _threefry2x32(0, 42, np.zeros(2, np.uint32),
                           np.arange(2, dtype=np.uint32))
    k0, k1 = c0[1], c1[1]
    n = L * U
    v0, v1 = _threefry2x32(k0, k1, np.zeros(n, np.uint32),
                           np.arange(n, dtype=np.uint32))
    idx = ((v0 ^ v1) % np.uint32(L)).astype(np.int64).reshape(L, U)
    c = np.zeros((L, L), np.int32)
    np.add.at(c, (np.arange(L)[:, None], idx), 1)
    return c.astype(np.int8)


_COUNTS = _sample_counts()


def _fused_kernel(x_ref, wq_ref, wk_ref, wv_ref, ct_ref, wfc_ref, bfc_ref,
                  gamma_ref, beta_ref, out_ref, qkv_scr, m_scr, mtop_scr,
                  bacc_ref, dr_scr):
    step = pl.program_id(0)

    @pl.when(step < HP)
    def _phase_aq():
        r = jax.lax.dot_general(x_ref[...], wq_ref[...],
                                (((1,), (0,)), ((), ())),
                                preferred_element_type=jnp.float32)
        qkv_scr[pl.ds(step, 1)] = (r * (1.0 / math.sqrt(DK)))[None]

    @pl.when(jnp.logical_and(step >= HP, step < 2 * HP))
    def _phase_ak():
        qkv_scr[pl.ds(step, 1)] = jax.lax.dot_general(
            x_ref[...], wk_ref[...], (((1,), (0,)), ((), ())),
            preferred_element_type=jnp.float32)[None]

    @pl.when(jnp.logical_and(step >= 2 * HP, step < NP))
    def _phase_av():
        qkv_scr[pl.ds(step, 1)] = jax.lax.dot_general(
            x_ref[...], wv_ref[...], (((1,), (0,)), ((), ())),
            preferred_element_type=jnp.float32)[None]

    @pl.when(jnp.logical_and(step >= SB0, step < SC_STEP))
    def _phase_b():
        i = step - SB0
        cf = ct_ref[...].astype(jnp.float32)
        mask = cf > 0.0
        cols = []
        for h in range(H):
            p, lo = h // 2, DK * (h % 2)
            qb = qkv_scr[p, pl.ds(i * QB, QB), pl.ds(lo, DK)]
            kh = qkv_scr[HP + p, :, pl.ds(lo, DK)]
            st = jax.lax.dot_general(qb, kh, (((1,), (1,)), ((), ())),
                                     preferred_element_type=jnp.float32)
            mx = jnp.max(jnp.where(mask, st, NEG), axis=1, keepdims=True)
            ksum = jax.lax.dot_general(cf, kh, (((1,), (0,)), ((), ())),
                                       preferred_element_type=jnp.float32)
            sm = jnp.sum(qb * ksum, axis=1, keepdims=True)
            cols.append(mx - sm * (1.0 / L))
        m_scr[pl.ds(i, 1)] = jnp.concatenate(cols, axis=1)[None]

    @pl.when(step == SC_STEP)
    def _phase_c():
        vals = jnp.transpose(
            jnp.concatenate([m_scr[i] for i in range(NQB)], axis=0))
        iot = jax.lax.broadcasted_iota(jnp.int32, (H, L), 1)
        cols = []
        for _ in range(U):
            mx = jnp.max(vals, axis=1, keepdims=True)
            idx_t = jnp.min(jnp.where(vals == mx, iot, L), axis=1,
                            keepdims=True)
            cols.append(idx_t)
            vals = jnp.where(iot == idx_t, NEG, vals)
        mtop_scr[...] = jnp.concatenate(cols, axis=1)

    @pl.when(step >= SD0)
    def _phase_d():
        pd = step - SD0
        pq = qkv_scr[pl.ds(pd, 1)][0]
        pk = qkv_scr[pl.ds(HP + pd, 1)][0]
        pv = qkv_scr[pl.ds(2 * HP + pd, 1)][0]
        iot = jax.lax.broadcasted_iota(jnp.int32, (L, U), 0)
        bases, drow_list = [], []
        for half in range(2):
            lo = DK * half
            q = pq[:, lo:lo + DK]
            k = pk[:, lo:lo + DK]
            v = pv[:, lo:lo + DK]
            wfc_h = wfc_ref[0, lo:lo + DK, :]
            sel = mtop_scr[pl.ds(2 * pd + half, 1), :]
            pt = (iot == sel).astype(jnp.float32)
            qr = jax.lax.dot_general(pt, q, (((0,), (0,)), ((), ())),
                                     preferred_element_type=jnp.float32)
            scores = jax.lax.dot_general(qr, k, (((1,), (1,)), ((), ())),
                                         preferred_element_type=jnp.float32)
            smax = jnp.max(scores, axis=1, keepdims=True)
            e = jnp.exp(scores - smax)
            ev = jax.lax.dot_general(e, v, (((1,), (0,)), ((), ())),
                                     preferred_element_type=jnp.float32)
            upd = ev * (1.0 / jnp.sum(e, axis=1, keepdims=True))
            meanv = jnp.mean(v, axis=0, keepdims=True)
            cat = jnp.concatenate([meanv, upd - meanv], axis=0)
            proj = jax.lax.dot_general(cat, wfc_h, (((1,), (0,)), ((), ())),
                                       preferred_element_type=jnp.float32)
            bases.append(proj[0:1])
            drow_list.append(proj[1:1 + U])
        base = bases[0] + bases[1]
        dr_scr[pl.ds(2 * U * pd, 2 * U)] = jnp.concatenate(drow_list, axis=0)

        @pl.when(pd == 0)
        def _():
            bacc_ref[...] = bfc_ref[...] + base

        @pl.when(pd > 0)
        def _():
            bacc_ref[...] += base

        @pl.when(pd == HP - 1)
        def _():
            iot2 = jax.lax.broadcasted_iota(jnp.int32, (L, U), 0)
            pts = []
            for h in range(H):
                selh = mtop_scr[pl.ds(h, 1), :]
                pts.append((iot2 == selh).astype(jnp.float32))
            pt_all = jnp.concatenate(pts, axis=1)
            val = (x_ref[...] + bacc_ref[...]
                   + jax.lax.dot_general(pt_all, dr_scr[...],
                                         (((1,), (0,)), ((), ())),
                                         preferred_element_type=jnp.float32))
            mu = jnp.mean(val, axis=1, keepdims=True)
            d = val - mu
            var = jnp.mean(d * d, axis=1, keepdims=True)
            r = 1.0 / jnp.sqrt(var + EPS)
            out_ref[...] = d * r * gamma_ref[...] + beta_ref[...]


def kernel(hidden_states, Wq, Wk, Wv, Wfc, bfc, gamma, beta):
    x = hidden_states.reshape(L, D)

    out = pl.pallas_call(
        _fused_kernel,
        grid=(NSTEPS,),
        in_specs=[
            pl.BlockSpec((L, D), lambda s: (0, 0)),
            pl.BlockSpec((D, 2 * DK), lambda s: (0, jnp.clip(s, 0, HP - 1))),
            pl.BlockSpec((D, 2 * DK), lambda s: (0, jnp.clip(s - HP, 0,
                                                             HP - 1))),
            pl.BlockSpec((D, 2 * DK), lambda s: (0, jnp.clip(s - 2 * HP, 0,
                                                             HP - 1))),
            pl.BlockSpec((QB, L), lambda s: (jnp.clip(s - SB0, 0, NQB - 1),
                                             0)),
            pl.BlockSpec((1, 2 * DK, D), lambda s: (jnp.clip(s - SD0, 0,
                                                             HP - 1), 0, 0)),
            pl.BlockSpec((1, D), lambda s: (0, 0)),
            pl.BlockSpec((1, D), lambda s: (0, 0)),
            pl.BlockSpec((1, D), lambda s: (0, 0)),
        ],
        out_specs=pl.BlockSpec((L, D), lambda s: (0, 0)),
        out_shape=jax.ShapeDtypeStruct((L, D), jnp.float32),
        scratch_shapes=[
            pltpu.VMEM((3 * HP, L, 2 * DK), jnp.float32),
            pltpu.VMEM((NQB, QB, H), jnp.float32),
            pltpu.VMEM((H, U), jnp.int32),
            pltpu.VMEM((1, D), jnp.float32),
            pltpu.VMEM((H * U, D), jnp.float32),
        ],
        interpret=INTERPRET,
    )(x, Wq, Wk, Wv, jnp.asarray(_COUNTS), Wfc.reshape(HP, 2 * DK, D),
      bfc.reshape(1, D), gamma.reshape(1, D), beta.reshape(1, D))

    return out.reshape(1, L, D)


# speedup vs baseline: 1.2342x; 1.2342x over previous
import math

import jax
import jax.numpy as jnp
import numpy as np
from jax.experimental import pallas as pl
from jax.experimental.pallas import tpu as pltpu

L = 2048
D = 768
H = 12
DK = 64
U = 40
NQB = 8
QB = L // NQB
EPS = 1e-6
NEG = -3.4e38

NP = 3 * H // 2
HP = H // 2
SB0 = NP
SC_STEP = NP + NQB
SD0 = SC_STEP + 1
NSTEPS = SD0 + HP

INTERPRET = False


def _threefry2x32(k0, k1, x0, x1):
    def rotl(x, d):
        return ((x << np.uint32(d)) | (x >> np.uint32(32 - d))).astype(np.uint32)
    rot = [13, 15, 26, 6, 17, 29, 16, 24]
    ks = [np.uint32(k0), np.uint32(k1),
          np.uint32(k0) ^ np.uint32(k1) ^ np.uint32(0x1BD11BDA)]
    x0 = (x0 + ks[0]).astype(np.uint32)
    x1 = (x1 + ks[1]).astype(np.uint32)
    for g in range(5):
        for j in range(4):
            x0 = (x0 + x1).astype(np.uint32)
            x1 = rotl(x1, rot[(g * 4 + j) % 8])
            x1 = x1 ^ x0
        x0 = (x0 + ks[(g + 1) % 3]).astype(np.uint32)
        x1 = (x1 + ks[(g + 2) % 3] + np.uint32(g + 1)).astype(np.uint32)
    return x0, x1


def _sample_counts() -> np.ndarray:
    c0, c1 = _threefry2x32(0, 42, np.zeros(2, np.uint32),
                           np.arange(2, dtype=np.uint32))
    k0, k1 = c0[1], c1[1]
    n = L * U
    v0, v1 = _threefry2x32(k0, k1, np.zeros(n, np.uint32),
                           np.arange(n, dtype=np.uint32))
    idx = ((v0 ^ v1) % np.uint32(L)).astype(np.int64).reshape(L, U)
    c = np.zeros((L, L), np.int32)
    np.add.at(c, (np.arange(L)[:, None], idx), 1)
    return np.ascontiguousarray(c.T).astype(np.int8)


_COUNTS = _sample_counts()


def _fused_kernel(x_ref, wq_ref, wk_ref, wv_ref, ct_ref, wfc_ref, bfc_ref,
                  gamma_ref, beta_ref, out_ref, qkv_scr, m_scr, mtop_scr,
                  bacc_ref, dr_scr):
    step = pl.program_id(0)

    @pl.when(step < HP)
    def _phase_aq():
        r = jax.lax.dot_general(x_ref[...], wq_ref[...],
                                (((1,), (0,)), ((), ())),
                                preferred_element_type=jnp.float32)
        qkv_scr[pl.ds(step, 1)] = (r * (1.0 / math.sqrt(DK)))[None]

    @pl.when(jnp.logical_and(step >= HP, step < 2 * HP))
    def _phase_ak():
        qkv_scr[pl.ds(step, 1)] = jax.lax.dot_general(
            x_ref[...], wk_ref[...], (((1,), (0,)), ((), ())),
            preferred_element_type=jnp.float32)[None]

    @pl.when(jnp.logical_and(step >= 2 * HP, step < NP))
    def _phase_av():
        qkv_scr[pl.ds(step, 1)] = jax.lax.dot_general(
            x_ref[...], wv_ref[...], (((1,), (0,)), ((), ())),
            preferred_element_type=jnp.float32)[None]

    @pl.when(jnp.logical_and(step >= SB0, step < SC_STEP))
    def _phase_b():
        i = step - SB0
        cf = ct_ref[...].astype(jnp.float32)
        mask = cf > 0.0
        rows = []
        for h in range(H):
            p, lo = h // 2, DK * (h % 2)
            qb = qkv_scr[p, pl.ds(i * QB, QB), pl.ds(lo, DK)]
            kh = qkv_scr[HP + p, :, pl.ds(lo, DK)]
            st = jax.lax.dot_general(kh, qb, (((1,), (1,)), ((), ())),
                                     preferred_element_type=jnp.float32)
            mx = jnp.max(jnp.where(mask, st, NEG), axis=0, keepdims=True)
            sm = jnp.sum(cf * st, axis=0, keepdims=True)
            rows.append(mx - sm * (1.0 / L))
        m_scr[pl.ds(i, 1)] = jnp.concatenate(rows, axis=0)[None]

    @pl.when(step == SC_STEP)
    def _phase_c():
        vals = jnp.concatenate([m_scr[i] for i in range(NQB)], axis=1)
        iot = jax.lax.broadcasted_iota(jnp.int32, (H, L), 1)
        cols = []
        for _ in range(U):
            mx = jnp.max(vals, axis=1, keepdims=True)
            idx_t = jnp.min(jnp.where(vals == mx, iot, L), axis=1,
                            keepdims=True)
            cols.append(idx_t)
            vals = jnp.where(iot == idx_t, NEG, vals)
        mtop_scr[...] = jnp.concatenate(cols, axis=1)

    @pl.when(step >= SD0)
    def _phase_d():
        pd = step - SD0
        pq = qkv_scr[pl.ds(pd, 1)][0]
        pk = qkv_scr[pl.ds(HP + pd, 1)][0]
        pv = qkv_scr[pl.ds(2 * HP + pd, 1)][0]
        iot = jax.lax.broadcasted_iota(jnp.int32, (L, U), 0)
        bases, drow_list = [], []
        for half in range(2):
            lo = DK * half
            q = pq[:, lo:lo + DK]
            k = pk[:, lo:lo + DK]
            v = pv[:, lo:lo + DK]
            wfc_h = wfc_ref[0, lo:lo + DK, :]
            sel = mtop_scr[pl.ds(2 * pd + half, 1), :]
            pt = (iot == sel).astype(jnp.float32)
            qr = jax.lax.dot_general(pt, q, (((0,), (0,)), ((), ())),
                                     preferred_element_type=jnp.float32)
            scores = jax.lax.dot_general(qr, k, (((1,), (1,)), ((), ())),
                                         preferred_element_type=jnp.float32)
            smax = jnp.max(scores, axis=1, keepdims=True)
            e = jnp.exp(scores - smax)
            ev = jax.lax.dot_general(e, v, (((1,), (0,)), ((), ())),
                                     preferred_element_type=jnp.float32)
            upd = ev * (1.0 / jnp.sum(e, axis=1, keepdims=True))
            meanv = jnp.mean(v, axis=0, keepdims=True)
            cat = jnp.concatenate([meanv, upd - meanv], axis=0)
            proj = jax.lax.dot_general(cat, wfc_h, (((1,), (0,)), ((), ())),
                                       preferred_element_type=jnp.float32)
            bases.append(proj[0:1])
            drow_list.append(proj[1:1 + U])
        base = bases[0] + bases[1]
        dr_scr[pl.ds(2 * U * pd, 2 * U)] = jnp.concatenate(drow_list, axis=0)

        @pl.when(pd == 0)
        def _():
            bacc_ref[...] = bfc_ref[...] + base

        @pl.when(pd > 0)
        def _():
            bacc_ref[...] += base

        @pl.when(pd == HP - 1)
        def _():
            iot2 = jax.lax.broadcasted_iota(jnp.int32, (L, U), 0)
            pts = []
            for h in range(H):
                selh = mtop_scr[pl.ds(h, 1), :]
                pts.append((iot2 == selh).astype(jnp.float32))
            pt_all = jnp.concatenate(pts, axis=1)
            val = (x_ref[...] + bacc_ref[...]
                   + jax.lax.dot_general(pt_all, dr_scr[...],
                                         (((1,), (0,)), ((), ())),
                                         preferred_element_type=jnp.float32))
            mu = jnp.mean(val, axis=1, keepdims=True)
            d = val - mu
            var = jnp.mean(d * d, axis=1, keepdims=True)
            r = 1.0 / jnp.sqrt(var + EPS)
            out_ref[...] = d * r * gamma_ref[...] + beta_ref[...]


def kernel(hidden_states, Wq, Wk, Wv, Wfc, bfc, gamma, beta):
    x = hidden_states.reshape(L, D)

    out = pl.pallas_call(
        _fused_kernel,
        grid=(NSTEPS,),
        in_specs=[
            pl.BlockSpec((L, D), lambda s: (0, 0)),
            pl.BlockSpec((D, 2 * DK), lambda s: (0, jnp.clip(s, 0, HP - 1))),
            pl.BlockSpec((D, 2 * DK), lambda s: (0, jnp.clip(s - HP, 0,
                                                             HP - 1))),
            pl.BlockSpec((D, 2 * DK), lambda s: (0, jnp.clip(s - 2 * HP, 0,
                                                             HP - 1))),
            pl.BlockSpec((L, QB), lambda s: (0, jnp.clip(s - SB0, 0,
                                                         NQB - 1))),
            pl.BlockSpec((1, 2 * DK, D), lambda s: (jnp.clip(s - SD0, 0,
                                                             HP - 1), 0, 0)),
            pl.BlockSpec((1, D), lambda s: (0, 0)),
            pl.BlockSpec((1, D), lambda s: (0, 0)),
            pl.BlockSpec((1, D), lambda s: (0, 0)),
        ],
        out_specs=pl.BlockSpec((L, D), lambda s: (0, 0)),
        out_shape=jax.ShapeDtypeStruct((L, D), jnp.float32),
        scratch_shapes=[
            pltpu.VMEM((3 * HP, L, 2 * DK), jnp.float32),
            pltpu.VMEM((NQB, H, QB), jnp.float32),
            pltpu.VMEM((H, U), jnp.int32),
            pltpu.VMEM((1, D), jnp.float32),
            pltpu.VMEM((H * U, D), jnp.float32),
        ],
        interpret=INTERPRET,
    )(x, Wq, Wk, Wv, jnp.asarray(_COUNTS), Wfc.reshape(HP, 2 * DK, D),
      bfc.reshape(1, D), gamma.reshape(1, D), beta.reshape(1, D))

    return out.reshape(1, L, D)


# speedup vs baseline: 1.3828x; 1.1204x over previous
import math

import jax
import jax.numpy as jnp
import numpy as np
from jax.experimental import pallas as pl
from jax.experimental.pallas import tpu as pltpu

L = 2048
D = 768
H = 12
DK = 64
U = 40
NQB = 8
QB = L // NQB
EPS = 1e-6
NEG = -3.4e38

NG = 4
NP = 3 * H // NG
HG = H // NG
SB0 = NP
SC_STEP = NP + NQB
SD0 = SC_STEP + 1
NSTEPS = SD0 + HG

INTERPRET = False


def _threefry2x32(k0, k1, x0, x1):
    def rotl(x, d):
        return ((x << np.uint32(d)) | (x >> np.uint32(32 - d))).astype(np.uint32)
    rot = [13, 15, 26, 6, 17, 29, 16, 24]
    ks = [np.uint32(k0), np.uint32(k1),
          np.uint32(k0) ^ np.uint32(k1) ^ np.uint32(0x1BD11BDA)]
    x0 = (x0 + ks[0]).astype(np.uint32)
    x1 = (x1 + ks[1]).astype(np.uint32)
    for g in range(5):
        for j in range(4):
            x0 = (x0 + x1).astype(np.uint32)
            x1 = rotl(x1, rot[(g * 4 + j) % 8])
            x1 = x1 ^ x0
        x0 = (x0 + ks[(g + 1) % 3]).astype(np.uint32)
        x1 = (x1 + ks[(g + 2) % 3] + np.uint32(g + 1)).astype(np.uint32)
    return x0, x1


def _sample_counts() -> np.ndarray:
    c0, c1 = _threefry2x32(0, 42, np.zeros(2, np.uint32),
                           np.arange(2, dtype=np.uint32))
    k0, k1 = c0[1], c1[1]
    n = L * U
    v0, v1 = _threefry2x32(k0, k1, np.zeros(n, np.uint32),
                           np.arange(n, dtype=np.uint32))
    idx = ((v0 ^ v1) % np.uint32(L)).astype(np.int64).reshape(L, U)
    c = np.zeros((L, L), np.int32)
    np.add.at(c, (np.arange(L)[:, None], idx), 1)
    return np.ascontiguousarray(c.T).astype(np.int8)


_COUNTS = _sample_counts()


def _fused_kernel(x_ref, wq_ref, wk_ref, wv_ref, ct_ref, wfc_ref, bfc_ref,
                  gamma_ref, beta_ref, out_ref, qkv_scr, m_scr, mtop_scr,
                  bacc_ref, dr_scr):
    step = pl.program_id(0)

    @pl.when(step < HG)
    def _phase_aq():
        r = jax.lax.dot_general(x_ref[...], wq_ref[...],
                                (((1,), (0,)), ((), ())),
                                preferred_element_type=jnp.float32)
        qkv_scr[pl.ds(step, 1)] = (r * (1.0 / math.sqrt(DK)))[None]

    @pl.when(jnp.logical_and(step >= HG, step < 2 * HG))
    def _phase_ak():
        qkv_scr[pl.ds(step, 1)] = jax.lax.dot_general(
            x_ref[...], wk_ref[...], (((1,), (0,)), ((), ())),
            preferred_element_type=jnp.float32)[None]

    @pl.when(jnp.logical_and(step >= 2 * HG, step < NP))
    def _phase_av():
        qkv_scr[pl.ds(step, 1)] = jax.lax.dot_general(
            x_ref[...], wv_ref[...], (((1,), (0,)), ((), ())),
            preferred_element_type=jnp.float32)[None]

    @pl.when(jnp.logical_and(step >= SB0, step < SC_STEP))
    def _phase_b():
        i = step - SB0
        cf = ct_ref[...].astype(jnp.float32)
        mask = cf > 0.0
        rows = []
        for h in range(H):
            p, lo = h // NG, DK * (h % NG)
            qb = qkv_scr[p, pl.ds(i * QB, QB), pl.ds(lo, DK)]
            kh = qkv_scr[HG + p, :, pl.ds(lo, DK)]
            st = jax.lax.dot_general(kh, qb, (((1,), (1,)), ((), ())),
                                     preferred_element_type=jnp.float32)
            mx = jnp.max(jnp.where(mask, st, NEG), axis=0, keepdims=True)
            sm = jnp.sum(cf * st, axis=0, keepdims=True)
            rows.append(mx - sm * (1.0 / L))
        m_scr[pl.ds(i, 1)] = jnp.concatenate(rows, axis=0)[None]

    @pl.when(step == SC_STEP)
    def _phase_c():
        vals = jnp.concatenate([m_scr[i] for i in range(NQB)], axis=1)
        iot = jax.lax.broadcasted_iota(jnp.int32, (H, L), 1)
        cols = []
        for _ in range(U):
            mx = jnp.max(vals, axis=1, keepdims=True)
            idx_t = jnp.min(jnp.where(vals == mx, iot, L), axis=1,
                            keepdims=True)
            cols.append(idx_t)
            vals = jnp.where(iot == idx_t, NEG, vals)
        mtop_scr[...] = jnp.concatenate(cols, axis=1)

    @pl.when(step >= SD0)
    def _phase_d():
        pd = step - SD0
        pq = qkv_scr[pl.ds(pd, 1)][0]
        pk = qkv_scr[pl.ds(HG + pd, 1)][0]
        pv = qkv_scr[pl.ds(2 * HG + pd, 1)][0]
        iot = jax.lax.broadcasted_iota(jnp.int32, (L, U), 0)
        bases, drow_list = [], []
        for half in range(NG):
            lo = DK * half
            q = pq[:, lo:lo + DK]
            k = pk[:, lo:lo + DK]
            v = pv[:, lo:lo + DK]
            wfc_h = wfc_ref[0, lo:lo + DK, :]
            sel = mtop_scr[pl.ds(NG * pd + half, 1), :]
            pt = (iot == sel).astype(jnp.float32)
            qr = jax.lax.dot_general(pt, q, (((0,), (0,)), ((), ())),
                                     preferred_element_type=jnp.float32)
            scores = jax.lax.dot_general(qr, k, (((1,), (1,)), ((), ())),
                                         preferred_element_type=jnp.float32)
            smax = jnp.max(scores, axis=1, keepdims=True)
            e = jnp.exp(scores - smax)
            ev = jax.lax.dot_general(e, v, (((1,), (0,)), ((), ())),
                                     preferred_element_type=jnp.float32)
            upd = ev * (1.0 / jnp.sum(e, axis=1, keepdims=True))
            meanv = jnp.mean(v, axis=0, keepdims=True)
            cat = jnp.concatenate([meanv, upd - meanv], axis=0)
            proj = jax.lax.dot_general(cat, wfc_h, (((1,), (0,)), ((), ())),
                                       preferred_element_type=jnp.float32)
            bases.append(proj[0:1])
            drow_list.append(proj[1:1 + U])
        base = sum(bases[1:], bases[0])
        dr_scr[pl.ds(NG * U * pd, NG * U)] = jnp.concatenate(drow_list, axis=0)

        @pl.when(pd == 0)
        def _():
            bacc_ref[...] = bfc_ref[...] + base

        @pl.when(pd > 0)
        def _():
            bacc_ref[...] += base

        @pl.when(pd == HG - 1)
        def _():
            iot2 = jax.lax.broadcasted_iota(jnp.int32, (L, U), 0)
            pts = []
            for h in range(H):
                selh = mtop_scr[pl.ds(h, 1), :]
                pts.append((iot2 == selh).astype(jnp.float32))
            pt_all = jnp.concatenate(pts, axis=1)
            val = (x_ref[...] + bacc_ref[...]
                   + jax.lax.dot_general(pt_all, dr_scr[...],
                                         (((1,), (0,)), ((), ())),
                                         preferred_element_type=jnp.float32))
            mu = jnp.mean(val, axis=1, keepdims=True)
            d = val - mu
            var = jnp.mean(d * d, axis=1, keepdims=True)
            r = 1.0 / jnp.sqrt(var + EPS)
            out_ref[...] = d * r * gamma_ref[...] + beta_ref[...]


def kernel(hidden_states, Wq, Wk, Wv, Wfc, bfc, gamma, beta):
    x = hidden_states.reshape(L, D)

    out = pl.pallas_call(
        _fused_kernel,
        grid=(NSTEPS,),
        in_specs=[
            pl.BlockSpec((L, D), lambda s: (0, 0)),
            pl.BlockSpec((D, NG * DK), lambda s: (0, jnp.clip(s, 0, HG - 1))),
            pl.BlockSpec((D, NG * DK), lambda s: (0, jnp.clip(s - HG, 0,
                                                             HG - 1))),
            pl.BlockSpec((D, NG * DK), lambda s: (0, jnp.clip(s - 2 * HG, 0,
                                                             HG - 1))),
            pl.BlockSpec((L, QB), lambda s: (0, jnp.clip(s - SB0, 0,
                                                         NQB - 1))),
            pl.BlockSpec((1, NG * DK, D), lambda s: (jnp.clip(s - SD0, 0,
                                                              HG - 1), 0, 0)),
            pl.BlockSpec((1, D), lambda s: (0, 0)),
            pl.BlockSpec((1, D), lambda s: (0, 0)),
            pl.BlockSpec((1, D), lambda s: (0, 0)),
        ],
        out_specs=pl.BlockSpec((L, D), lambda s: (0, 0)),
        out_shape=jax.ShapeDtypeStruct((L, D), jnp.float32),
        scratch_shapes=[
            pltpu.VMEM((3 * HG, L, NG * DK), jnp.float32),
            pltpu.VMEM((NQB, H, QB), jnp.float32),
            pltpu.VMEM((H, U), jnp.int32),
            pltpu.VMEM((1, D), jnp.float32),
            pltpu.VMEM((H * U, D), jnp.float32),
        ],
        interpret=INTERPRET,
    )(x, Wq, Wk, Wv, jnp.asarray(_COUNTS), Wfc.reshape(HG, NG * DK, D),
      bfc.reshape(1, D), gamma.reshape(1, D), beta.reshape(1, D))

    return out.reshape(1, L, D)


# speedup vs baseline: 1.4124x; 1.0214x over previous
import math

import jax
import jax.numpy as jnp
import numpy as np
from jax.experimental import pallas as pl
from jax.experimental.pallas import tpu as pltpu

L = 2048
D = 768
H = 12
DK = 64
U = 40
NQB = 8
QB = L // NQB
EPS = 1e-6
NEG = -3.4e38

NG = 4
NP = 3
HG = H // NG
SB0 = NP
SC_STEP = NP + NQB
SD0 = SC_STEP + 1
NSTEPS = SD0 + HG

INTERPRET = False


def _threefry2x32(k0, k1, x0, x1):
    def rotl(x, d):
        return ((x << np.uint32(d)) | (x >> np.uint32(32 - d))).astype(np.uint32)
    rot = [13, 15, 26, 6, 17, 29, 16, 24]
    ks = [np.uint32(k0), np.uint32(k1),
          np.uint32(k0) ^ np.uint32(k1) ^ np.uint32(0x1BD11BDA)]
    x0 = (x0 + ks[0]).astype(np.uint32)
    x1 = (x1 + ks[1]).astype(np.uint32)
    for g in range(5):
        for j in range(4):
            x0 = (x0 + x1).astype(np.uint32)
            x1 = rotl(x1, rot[(g * 4 + j) % 8])
            x1 = x1 ^ x0
        x0 = (x0 + ks[(g + 1) % 3]).astype(np.uint32)
        x1 = (x1 + ks[(g + 2) % 3] + np.uint32(g + 1)).astype(np.uint32)
    return x0, x1


def _sample_counts() -> np.ndarray:
    c0, c1 = _threefry2x32(0, 42, np.zeros(2, np.uint32),
                           np.arange(2, dtype=np.uint32))
    k0, k1 = c0[1], c1[1]
    n = L * U
    v0, v1 = _threefry2x32(k0, k1, np.zeros(n, np.uint32),
                           np.arange(n, dtype=np.uint32))
    idx = ((v0 ^ v1) % np.uint32(L)).astype(np.int64).reshape(L, U)
    c = np.zeros((L, L), np.int32)
    np.add.at(c, (np.arange(L)[:, None], idx), 1)
    return np.ascontiguousarray(c.T).astype(np.int8)


_COUNTS = _sample_counts()


def _fused_kernel(x_ref, wq_ref, wk_ref, wv_ref, ct_ref, wfc_ref, bfc_ref,
                  gamma_ref, beta_ref, out_ref, qkv_scr, m_scr, mtop_scr,
                  bacc_ref, dr_scr):
    step = pl.program_id(0)

    @pl.when(step == 0)
    def _phase_aq():
        r = jax.lax.dot_general(x_ref[...], wq_ref[...],
                                (((1,), (0,)), ((), ())),
                                preferred_element_type=jnp.float32)
        r = r * (1.0 / math.sqrt(DK))
        for g in range(HG):
            qkv_scr[g] = r[:, NG * DK * g:NG * DK * (g + 1)]

    @pl.when(step == 1)
    def _phase_ak():
        r = jax.lax.dot_general(x_ref[...], wk_ref[...],
                                (((1,), (0,)), ((), ())),
                                preferred_element_type=jnp.float32)
        for g in range(HG):
            qkv_scr[HG + g] = r[:, NG * DK * g:NG * DK * (g + 1)]

    @pl.when(step == 2)
    def _phase_av():
        r = jax.lax.dot_general(x_ref[...], wv_ref[...],
                                (((1,), (0,)), ((), ())),
                                preferred_element_type=jnp.float32)
        for g in range(HG):
            qkv_scr[2 * HG + g] = r[:, NG * DK * g:NG * DK * (g + 1)]

    @pl.when(jnp.logical_and(step >= SB0, step < SC_STEP))
    def _phase_b():
        i = step - SB0
        cf = ct_ref[...].astype(jnp.float32)
        mask = cf > 0.0
        rows = []
        for h in range(H):
            p, lo = h // NG, DK * (h % NG)
            qb = qkv_scr[p, pl.ds(i * QB, QB), pl.ds(lo, DK)]
            kh = qkv_scr[HG + p, :, pl.ds(lo, DK)]
            st = jax.lax.dot_general(kh, qb, (((1,), (1,)), ((), ())),
                                     preferred_element_type=jnp.float32)
            mx = jnp.max(jnp.where(mask, st, NEG), axis=0, keepdims=True)
            sm = jnp.sum(cf * st, axis=0, keepdims=True)
            rows.append(mx - sm * (1.0 / L))
        m_scr[pl.ds(i, 1)] = jnp.concatenate(rows, axis=0)[None]

    @pl.when(step == SC_STEP)
    def _phase_c():
        vals = jnp.concatenate([m_scr[i] for i in range(NQB)], axis=1)
        iot = jax.lax.broadcasted_iota(jnp.int32, (H, L), 1)
        cols = []
        for _ in range(U):
            mx = jnp.max(vals, axis=1, keepdims=True)
            idx_t = jnp.min(jnp.where(vals == mx, iot, L), axis=1,
                            keepdims=True)
            cols.append(idx_t)
            vals = jnp.where(iot == idx_t, NEG, vals)
        mtop_scr[...] = jnp.concatenate(cols, axis=1)

    @pl.when(step >= SD0)
    def _phase_d():
        pd = step - SD0
        pq = qkv_scr[pl.ds(pd, 1)][0]
        pk = qkv_scr[pl.ds(HG + pd, 1)][0]
        pv = qkv_scr[pl.ds(2 * HG + pd, 1)][0]
        iot = jax.lax.broadcasted_iota(jnp.int32, (L, U), 0)
        bases, drow_list = [], []
        for half in range(NG):
            lo = DK * half
            q = pq[:, lo:lo + DK]
            k = pk[:, lo:lo + DK]
            v = pv[:, lo:lo + DK]
            wfc_h = wfc_ref[0, lo:lo + DK, :]
            sel = mtop_scr[pl.ds(NG * pd + half, 1), :]
            pt = (iot == sel).astype(jnp.float32)
            qr = jax.lax.dot_general(pt, q, (((0,), (0,)), ((), ())),
                                     preferred_element_type=jnp.float32)
            scores = jax.lax.dot_general(qr, k, (((1,), (1,)), ((), ())),
                                         preferred_element_type=jnp.float32)
            smax = jnp.max(scores, axis=1, keepdims=True)
            e = jnp.exp(scores - smax)
            ev = jax.lax.dot_general(e, v, (((1,), (0,)), ((), ())),
                                     preferred_element_type=jnp.float32)
            upd = ev * (1.0 / jnp.sum(e, axis=1, keepdims=True))
            meanv = jnp.mean(v, axis=0, keepdims=True)
            cat = jnp.concatenate([meanv, upd - meanv], axis=0)
            proj = jax.lax.dot_general(cat, wfc_h, (((1,), (0,)), ((), ())),
                                       preferred_element_type=jnp.float32)
            bases.append(proj[0:1])
            drow_list.append(proj[1:1 + U])
        base = sum(bases[1:], bases[0])
        dr_scr[pl.ds(NG * U * pd, NG * U)] = jnp.concatenate(drow_list, axis=0)

        @pl.when(pd == 0)
        def _():
            bacc_ref[...] = bfc_ref[...] + base

        @pl.when(pd > 0)
        def _():
            bacc_ref[...] += base

        @pl.when(pd == HG - 1)
        def _():
            iot2 = jax.lax.broadcasted_iota(jnp.int32, (L, U), 0)
            pts = []
            for h in range(H):
                selh = mtop_scr[pl.ds(h, 1), :]
                pts.append((iot2 == selh).astype(jnp.float32))
            pt_all = jnp.concatenate(pts, axis=1)
            val = (x_ref[...] + bacc_ref[...]
                   + jax.lax.dot_general(pt_all, dr_scr[...],
                                         (((1,), (0,)), ((), ())),
                                         preferred_element_type=jnp.float32))
            mu = jnp.mean(val, axis=1, keepdims=True)
            d = val - mu
            var = jnp.mean(d * d, axis=1, keepdims=True)
            r = 1.0 / jnp.sqrt(var + EPS)
            out_ref[...] = d * r * gamma_ref[...] + beta_ref[...]


def kernel(hidden_states, Wq, Wk, Wv, Wfc, bfc, gamma, beta):
    x = hidden_states.reshape(L, D)

    out = pl.pallas_call(
        _fused_kernel,
        grid=(NSTEPS,),
        in_specs=[
            pl.BlockSpec((L, D), lambda s: (0, 0)),
            pl.BlockSpec((D, D), lambda s: (0, 0)),
            pl.BlockSpec((D, D), lambda s: (0, 0)),
            pl.BlockSpec((D, D), lambda s: (0, 0)),
            pl.BlockSpec((L, QB), lambda s: (0, jnp.clip(s - SB0, 0,
                                                         NQB - 1))),
            pl.BlockSpec((1, NG * DK, D), lambda s: (jnp.clip(s - SD0, 0,
                                                              HG - 1), 0, 0)),
            pl.BlockSpec((1, D), lambda s: (0, 0)),
            pl.BlockSpec((1, D), lambda s: (0, 0)),
            pl.BlockSpec((1, D), lambda s: (0, 0)),
        ],
        out_specs=pl.BlockSpec((L, D), lambda s: (0, 0)),
        out_shape=jax.ShapeDtypeStruct((L, D), jnp.float32),
        scratch_shapes=[
            pltpu.VMEM((3 * HG, L, NG * DK), jnp.float32),
            pltpu.VMEM((NQB, H, QB), jnp.float32),
            pltpu.VMEM((H, U), jnp.int32),
            pltpu.VMEM((1, D), jnp.float32),
            pltpu.VMEM((H * U, D), jnp.float32),
        ],
        interpret=INTERPRET,
    )(x, Wq, Wk, Wv, jnp.asarray(_COUNTS), Wfc.reshape(HG, NG * DK, D),
      bfc.reshape(1, D), gamma.reshape(1, D), beta.reshape(1, D))

    return out.reshape(1, L, D)
